# baseline (speedup 1.0000x reference)
import jax
import jax.numpy as jnp
from jax import lax
from jax.experimental import pallas as pl
from jax.experimental.pallas import tpu as pltpu
from jax.experimental.pallas import tpu_sc as plsc

N_VENDORS = 1000000
N_WEEKS = 520
NW_PAD = 528
EMB = 16
B = 16384
NWORK = 32
BPW = B // NWORK
NCHUNK = 4
CHUNK = BPW // NCHUNK


def _fe_kernel(vt_hbm, wtp_hbm, vid_hbm, wk_hbm, lc_hbm, wv_hbm, ww_hbm,
               misc_hbm, out_hbm, vid_v, cols_v, wtab_v, wproj_v,
               wk_v, lc_v, wv_r, ww_r, misc_r, out_v, sem):
    nc = 2
    wid = lax.axis_index("s") * nc + lax.axis_index("c")
    iota = lax.iota(jnp.int32, EMB)

    pltpu.sync_copy(vid_hbm.at[pl.ds(wid * NCHUNK, NCHUNK)], vid_v)

    copies = []
    for k in range(EMB):
        for j in range(NCHUNK):
            copies.append(pltpu.async_copy(
                vt_hbm.at[k].at[vid_v.at[j]], cols_v.at[k, j], sem))

    pltpu.sync_copy(wk_hbm.at[pl.ds(wid * NCHUNK, NCHUNK)], wk_v)
    pltpu.sync_copy(lc_hbm.at[pl.ds(wid * NCHUNK, NCHUNK)], lc_v)
    pltpu.sync_copy(wtp_hbm, wtab_v)
    pltpu.sync_copy(wv_hbm, wv_r)
    pltpu.sync_copy(ww_hbm, ww_r)
    pltpu.sync_copy(misc_hbm, misc_r)

    scale = misc_r[0] * misc_r[1]
    bias = misc_r[2]

    for c in range(NW_PAD // EMB):
        sl = pl.ds(c * EMB, EMB)
        acc = wtab_v[0, sl] * ww_r[0]
        for k in range(1, EMB):
            acc = acc + wtab_v[k, sl] * ww_r[k]
        wproj_v[c] = acc

    for cp in copies:
        cp.wait()

    def body(j, carry):
        for c in range(CHUNK // EMB):
            sl = pl.ds(c * EMB, EMB)
            wid_vec = wk_v[j, sl]
            acc0 = lc_v[j, sl] * scale + bias
            acc0 = acc0 + plsc.load_gather(
                wproj_v, [lax.shift_right_logical(wid_vec, 4),
                          lax.bitwise_and(wid_vec, 15)])
            acc1 = cols_v[0, j, sl] * wv_r[0]
            acc2 = cols_v[1, j, sl] * wv_r[1]
            acc3 = cols_v[2, j, sl] * wv_r[2]
            accs = [acc0, acc1, acc2, acc3]
            for k in range(3, EMB):
                a = accs[(k + 1) % 4]
                accs[(k + 1) % 4] = a + cols_v[k, j, sl] * wv_r[k]
            out_v[j, sl] = (accs[0] + accs[1]) + (accs[2] + accs[3])
        return carry

    lax.fori_loop(0, NCHUNK, body, 0)
    pltpu.sync_copy(out_v, out_hbm.at[pl.ds(wid * NCHUNK, NCHUNK)])


@jax.jit
def kernel(vendor_ids, week_ids, log_clicks, vendor_emb, week_emb, click_w,
           comb_W, comb_b):
    vt = vendor_emb.T
    wtp = jnp.pad(week_emb.T, ((0, 0), (0, NW_PAD - N_WEEKS)))
    vid = vendor_ids.reshape(NWORK * NCHUNK, CHUNK)
    wk = week_ids.reshape(NWORK * NCHUNK, CHUNK)
    lc = log_clicks.reshape(NWORK * NCHUNK, CHUNK)
    wv = jnp.broadcast_to(comb_W[0, 0:EMB, None], (EMB, EMB))
    ww = jnp.broadcast_to(comb_W[0, EMB:2 * EMB, None], (EMB, EMB))
    misc = jnp.broadcast_to(
        jnp.concatenate([comb_W[0, 2 * EMB:], click_w[0], comb_b])[:, None],
        (3, EMB))

    mesh = plsc.VectorSubcoreMesh(core_axis_name="c", subcore_axis_name="s")
    run = pl.kernel(
        _fe_kernel, mesh=mesh,
        compiler_params=pltpu.CompilerParams(
            needs_layout_passes=False, use_tc_tiling_on_sc=False),
        out_type=jax.ShapeDtypeStruct((NWORK * NCHUNK, CHUNK), jnp.float32),
        scratch_types=[
            pltpu.VMEM((NCHUNK, CHUNK), jnp.int32),
            pltpu.VMEM((EMB, NCHUNK, CHUNK), jnp.float32),
            pltpu.VMEM((EMB, NW_PAD), jnp.float32),
            pltpu.VMEM((NW_PAD // EMB, EMB), jnp.float32),
            pltpu.VMEM((NCHUNK, CHUNK), jnp.int32),
            pltpu.VMEM((NCHUNK, CHUNK), jnp.float32),
            pltpu.VMEM((EMB, EMB), jnp.float32),
            pltpu.VMEM((EMB, EMB), jnp.float32),
            pltpu.VMEM((3, EMB), jnp.float32),
            pltpu.VMEM((NCHUNK, CHUNK), jnp.float32),
            pltpu.SemaphoreType.DMA,
        ],
    )
    out = run(vt, wtp, vid, wk, lc, wv, ww, misc)
    return out.reshape(B)


# speedup vs baseline: 10.4500x; 10.4500x over previous
import jax
import jax.numpy as jnp
from jax import lax
from jax.experimental import pallas as pl
from jax.experimental.pallas import tpu as pltpu
from jax.experimental.pallas import tpu_sc as plsc

N_VENDORS = 1000000
N_WEEKS = 520
NW_PAD = 528
EMB = 16
B = 16384
NWORK = 32
BPW = B // NWORK
NCHUNK = 4
CHUNK = BPW // NCHUNK

VB_FULL = N_VENDORS // 128
VB_PER_W = 248
GRP = 8


def _detile(vt_hbm, out_hbm, bufs, rag, sem_in, sem_out, sem_rag):
    nc = 2
    w = lax.axis_index("s") * nc + lax.axis_index("c")
    base = w * VB_PER_W
    last = VB_FULL - 1

    def body(g, carry):
        n0 = base + g * GRP
        ins = []
        for u in range(GRP):
            vb = jnp.minimum(n0 + u, last)
            for kh in range(2):
                ins.append(pltpu.async_copy(
                    vt_hbm.at[pl.ds(8 * kh, 8), pl.ds(vb * 128, 128)],
                    bufs.at[u, kh], sem_in))
        for cp in ins:
            cp.wait()
        outs = []
        for u in range(GRP):
            vb = jnp.minimum(n0 + u, last)
            col = vb * 128
            for kh in range(2):
                for kl in range(8):
                    k = 8 * kh + kl
                    outs.append(pltpu.async_copy(
                        bufs.at[u, kh, kl],
                        out_hbm.at[pl.ds(k * N_VENDORS + col, 128)],
                        sem_out))
        for cp in outs:
            cp.wait()
        return carry

    lax.fori_loop(0, VB_PER_W // GRP, body, 0)

    @pl.when(w == 0)
    def _():
        cps = []
        for kh in range(2):
            cps.append(pltpu.async_copy(
                vt_hbm.at[pl.ds(8 * kh, 8), pl.ds(VB_FULL * 128, 64)],
                rag.at[kh], sem_rag))
        for cp in cps:
            cp.wait()
        cps = []
        for kh in range(2):
            for kl in range(8):
                k = 8 * kh + kl
                cps.append(pltpu.async_copy(
                    rag.at[kh, kl],
                    out_hbm.at[pl.ds(k * N_VENDORS + VB_FULL * 128, 64)],
                    sem_rag))
        for cp in cps:
            cp.wait()


def _gather(vt_hbm, wtp_hbm, vid_hbm, wk_hbm, lc_hbm, wv_hbm, ww_hbm,
            misc_hbm, out_hbm, vid_v, cols_v, wtab_v, wproj_v,
            wk_v, lc_v, wv_r, ww_r, misc_r, out_v, sem):
    nc = 2
    wid = lax.axis_index("s") * nc + lax.axis_index("c")

    pltpu.sync_copy(vid_hbm.at[pl.ds(wid * NCHUNK, NCHUNK)], vid_v)

    copies = []
    for k in range(EMB):
        for j in range(NCHUNK):
            copies.append(pltpu.async_copy(
                vt_hbm.at[k].at[vid_v.at[j]], cols_v.at[k, j], sem))

    pltpu.sync_copy(wk_hbm.at[pl.ds(wid * NCHUNK, NCHUNK)], wk_v)
    pltpu.sync_copy(lc_hbm.at[pl.ds(wid * NCHUNK, NCHUNK)], lc_v)
    pltpu.sync_copy(wtp_hbm, wtab_v)
    pltpu.sync_copy(wv_hbm, wv_r)
    pltpu.sync_copy(ww_hbm, ww_r)
    pltpu.sync_copy(misc_hbm, misc_r)

    scale = misc_r[0] * misc_r[1]
    bias = misc_r[2]

    for c in range(NW_PAD // EMB):
        sl = pl.ds(c * EMB, EMB)
        acc = wtab_v[0, sl] * ww_r[0]
        for k in range(1, EMB):
            acc = acc + wtab_v[k, sl] * ww_r[k]
        wproj_v[c] = acc

    for cp in copies:
        cp.wait()

    def body(j, carry):
        for c in range(CHUNK // EMB):
            sl = pl.ds(c * EMB, EMB)
            wid_vec = wk_v[j, sl]
            acc0 = lc_v[j, sl] * scale + bias
            acc0 = acc0 + plsc.load_gather(
                wproj_v, [lax.shift_right_logical(wid_vec, 4),
                          lax.bitwise_and(wid_vec, 15)])
            acc1 = cols_v[0, j, sl] * wv_r[0]
            acc2 = cols_v[1, j, sl] * wv_r[1]
            acc3 = cols_v[2, j, sl] * wv_r[2]
            accs = [acc0, acc1, acc2, acc3]
            for k in range(3, EMB):
                a = accs[(k + 1) % 4]
                accs[(k + 1) % 4] = a + cols_v[k, j, sl] * wv_r[k]
            out_v[j, sl] = (accs[0] + accs[1]) + (accs[2] + accs[3])
        return carry

    lax.fori_loop(0, NCHUNK, body, 0)
    pltpu.sync_copy(out_v, out_hbm.at[pl.ds(wid * NCHUNK, NCHUNK)])


@jax.jit
def kernel(vendor_ids, week_ids, log_clicks, vendor_emb, week_emb, click_w,
           comb_W, comb_b):
    vt = vendor_emb.T
    wtp = jnp.pad(week_emb.T, ((0, 0), (0, NW_PAD - N_WEEKS)))
    vid = vendor_ids.reshape(NWORK * NCHUNK, CHUNK)
    wk = week_ids.reshape(NWORK * NCHUNK, CHUNK)
    lc = log_clicks.reshape(NWORK * NCHUNK, CHUNK)
    wv = jnp.broadcast_to(comb_W[0, 0:EMB, None], (EMB, EMB))
    ww = jnp.broadcast_to(comb_W[0, EMB:2 * EMB, None], (EMB, EMB))
    misc = jnp.broadcast_to(
        jnp.concatenate([comb_W[0, 2 * EMB:], click_w[0], comb_b])[:, None],
        (3, EMB))

    mesh = plsc.VectorSubcoreMesh(core_axis_name="c", subcore_axis_name="s")
    detile = pl.kernel(
        _detile, mesh=mesh,
        compiler_params=pltpu.CompilerParams(
            needs_layout_passes=False, use_tc_tiling_on_sc=True),
        out_type=jax.ShapeDtypeStruct((EMB * N_VENDORS,), jnp.float32),
        scratch_types=[
            pltpu.VMEM((GRP, 2, 8, 128), jnp.float32),
            pltpu.VMEM((2, 8, 64), jnp.float32),
            pltpu.SemaphoreType.DMA,
            pltpu.SemaphoreType.DMA,
            pltpu.SemaphoreType.DMA,
        ],
    )
    vt_lin = detile(vt).reshape(EMB, N_VENDORS)

    mesh2 = plsc.VectorSubcoreMesh(core_axis_name="c", subcore_axis_name="s")
    gather = pl.kernel(
        _gather, mesh=mesh2,
        compiler_params=pltpu.CompilerParams(
            needs_layout_passes=False, use_tc_tiling_on_sc=False),
        out_type=jax.ShapeDtypeStruct((NWORK * NCHUNK, CHUNK), jnp.float32),
        scratch_types=[
            pltpu.VMEM((NCHUNK, CHUNK), jnp.int32),
            pltpu.VMEM((EMB, NCHUNK, CHUNK), jnp.float32),
            pltpu.VMEM((EMB, NW_PAD), jnp.float32),
            pltpu.VMEM((NW_PAD // EMB, EMB), jnp.float32),
            pltpu.VMEM((NCHUNK, CHUNK), jnp.int32),
            pltpu.VMEM((NCHUNK, CHUNK), jnp.float32),
            pltpu.VMEM((EMB, EMB), jnp.float32),
            pltpu.VMEM((EMB, EMB), jnp.float32),
            pltpu.VMEM((3, EMB), jnp.float32),
            pltpu.VMEM((NCHUNK, CHUNK), jnp.float32),
            pltpu.SemaphoreType.DMA,
        ],
    )
    out = gather(vt_lin, wtp, vid, wk, lc, wv, ww, misc)
    return out.reshape(B)


# speedup vs baseline: 11.7616x; 1.1255x over previous
import jax
import jax.numpy as jnp
from jax import lax
from jax.experimental import pallas as pl
from jax.experimental.pallas import tpu as pltpu
from jax.experimental.pallas import tpu_sc as plsc

N_VENDORS = 1000000
N_WEEKS = 520
NW_PAD = 528
EMB = 16
B = 16384
NWORK = 32
BPW = B // NWORK
NCHUNK = 4
CHUNK = BPW // NCHUNK

VB_FULL = N_VENDORS // 128
VB_PER_W = 248
GRP = 8


def _detile(vt_hbm, out_hbm, bufs, rag, sem_in, sem_out, sem_rag):
    nc = 2
    w = lax.axis_index("s") * nc + lax.axis_index("c")
    base = w * VB_PER_W
    last = VB_FULL - 1
    ngrp = VB_PER_W // GRP

    def fire_in(g, p):
        n0 = base + g * GRP
        for u in range(GRP):
            vb = jnp.minimum(n0 + u, last)
            for kh in range(2):
                pltpu.async_copy(
                    vt_hbm.at[pl.ds(8 * kh, 8), pl.ds(vb * 128, 128)],
                    bufs.at[p, u, kh], sem_in)

    def wait_in():
        for u in range(GRP):
            for kh in range(2):
                pltpu.make_async_copy(
                    vt_hbm.at[pl.ds(0, 8), pl.ds(0, 128)],
                    bufs.at[0, u, kh], sem_in).wait()

    fire_in(0, 0)

    def body(g, carry):
        p = lax.rem(g, 2)
        wait_in()
        fire_in(g + 1, 1 - p)
        n0 = base + g * GRP
        for u in range(GRP):
            vb = jnp.minimum(n0 + u, last)
            col = vb * 128
            for kh in range(2):
                for kl in range(8):
                    k = 8 * kh + kl
                    pltpu.async_copy(
                        bufs.at[p, u, kh, kl],
                        out_hbm.at[pl.ds(k * N_VENDORS + col, 128)],
                        sem_out)
        for u in range(GRP):
            for kh in range(2):
                for kl in range(8):
                    pltpu.make_async_copy(
                        out_hbm.at[pl.ds(0, 128)],
                        bufs.at[0, u, kh, kl], sem_out).wait()
        return carry

    lax.fori_loop(0, ngrp, body, 0)
    wait_in()

    @pl.when(w == 0)
    def _():
        cps = []
        for kh in range(2):
            cps.append(pltpu.async_copy(
                vt_hbm.at[pl.ds(8 * kh, 8), pl.ds(VB_FULL * 128, 64)],
                rag.at[kh], sem_rag))
        for cp in cps:
            cp.wait()
        cps = []
        for kh in range(2):
            for kl in range(8):
                k = 8 * kh + kl
                cps.append(pltpu.async_copy(
                    rag.at[kh, kl],
                    out_hbm.at[pl.ds(k * N_VENDORS + VB_FULL * 128, 64)],
                    sem_rag))
        for cp in cps:
            cp.wait()


def _gather(vt_hbm, wtp_hbm, vid_hbm, wk_hbm, lc_hbm, wv_hbm, ww_hbm,
            misc_hbm, out_hbm, vid_v, cols_v, wtab_v, wproj_v,
            wk_v, lc_v, wv_r, ww_r, misc_r, out_v, sem):
    nc = 2
    wid = lax.axis_index("s") * nc + lax.axis_index("c")

    pltpu.sync_copy(vid_hbm.at[pl.ds(wid * NCHUNK, NCHUNK)], vid_v)

    copies = []
    for k in range(EMB):
        for j in range(NCHUNK):
            copies.append(pltpu.async_copy(
                vt_hbm.at[k].at[vid_v.at[j]], cols_v.at[k, j], sem))

    pltpu.sync_copy(wk_hbm.at[pl.ds(wid * NCHUNK, NCHUNK)], wk_v)
    pltpu.sync_copy(lc_hbm.at[pl.ds(wid * NCHUNK, NCHUNK)], lc_v)
    pltpu.sync_copy(wtp_hbm, wtab_v)
    pltpu.sync_copy(wv_hbm, wv_r)
    pltpu.sync_copy(ww_hbm, ww_r)
    pltpu.sync_copy(misc_hbm, misc_r)

    scale = misc_r[0] * misc_r[1]
    bias = misc_r[2]

    for c in range(NW_PAD // EMB):
        sl = pl.ds(c * EMB, EMB)
        acc = wtab_v[0, sl] * ww_r[0]
        for k in range(1, EMB):
            acc = acc + wtab_v[k, sl] * ww_r[k]
        wproj_v[c] = acc

    for cp in copies:
        cp.wait()

    def body(j, carry):
        for c in range(CHUNK // EMB):
            sl = pl.ds(c * EMB, EMB)
            wid_vec = wk_v[j, sl]
            acc0 = lc_v[j, sl] * scale + bias
            acc0 = acc0 + plsc.load_gather(
                wproj_v, [lax.shift_right_logical(wid_vec, 4),
                          lax.bitwise_and(wid_vec, 15)])
            acc1 = cols_v[0, j, sl] * wv_r[0]
            acc2 = cols_v[1, j, sl] * wv_r[1]
            acc3 = cols_v[2, j, sl] * wv_r[2]
            accs = [acc0, acc1, acc2, acc3]
            for k in range(3, EMB):
                a = accs[(k + 1) % 4]
                accs[(k + 1) % 4] = a + cols_v[k, j, sl] * wv_r[k]
            out_v[j, sl] = (accs[0] + accs[1]) + (accs[2] + accs[3])
        return carry

    lax.fori_loop(0, NCHUNK, body, 0)
    pltpu.sync_copy(out_v, out_hbm.at[pl.ds(wid * NCHUNK, NCHUNK)])


@jax.jit
def kernel(vendor_ids, week_ids, log_clicks, vendor_emb, week_emb, click_w,
           comb_W, comb_b):
    vt = vendor_emb.T
    wtp = jnp.pad(week_emb.T, ((0, 0), (0, NW_PAD - N_WEEKS)))
    vid = vendor_ids.reshape(NWORK * NCHUNK, CHUNK)
    wk = week_ids.reshape(NWORK * NCHUNK, CHUNK)
    lc = log_clicks.reshape(NWORK * NCHUNK, CHUNK)
    wv = jnp.broadcast_to(comb_W[0, 0:EMB, None], (EMB, EMB))
    ww = jnp.broadcast_to(comb_W[0, EMB:2 * EMB, None], (EMB, EMB))
    misc = jnp.broadcast_to(
        jnp.concatenate([comb_W[0, 2 * EMB:], click_w[0], comb_b])[:, None],
        (3, EMB))

    mesh = plsc.VectorSubcoreMesh(core_axis_name="c", subcore_axis_name="s")
    detile = pl.kernel(
        _detile, mesh=mesh,
        compiler_params=pltpu.CompilerParams(
            needs_layout_passes=False, use_tc_tiling_on_sc=True),
        out_type=jax.ShapeDtypeStruct((EMB * N_VENDORS,), jnp.float32),
        scratch_types=[
            pltpu.VMEM((2, GRP, 2, 8, 128), jnp.float32),
            pltpu.VMEM((2, 8, 64), jnp.float32),
            pltpu.SemaphoreType.DMA,
            pltpu.SemaphoreType.DMA,
            pltpu.SemaphoreType.DMA,
        ],
    )
    vt_lin = detile(vt).reshape(EMB, N_VENDORS)

    mesh2 = plsc.VectorSubcoreMesh(core_axis_name="c", subcore_axis_name="s")
    gather = pl.kernel(
        _gather, mesh=mesh2,
        compiler_params=pltpu.CompilerParams(
            needs_layout_passes=False, use_tc_tiling_on_sc=False),
        out_type=jax.ShapeDtypeStruct((NWORK * NCHUNK, CHUNK), jnp.float32),
        scratch_types=[
            pltpu.VMEM((NCHUNK, CHUNK), jnp.int32),
            pltpu.VMEM((EMB, NCHUNK, CHUNK), jnp.float32),
            pltpu.VMEM((EMB, NW_PAD), jnp.float32),
            pltpu.VMEM((NW_PAD // EMB, EMB), jnp.float32),
            pltpu.VMEM((NCHUNK, CHUNK), jnp.int32),
            pltpu.VMEM((NCHUNK, CHUNK), jnp.float32),
            pltpu.VMEM((EMB, EMB), jnp.float32),
            pltpu.VMEM((EMB, EMB), jnp.float32),
            pltpu.VMEM((3, EMB), jnp.float32),
            pltpu.VMEM((NCHUNK, CHUNK), jnp.float32),
            pltpu.SemaphoreType.DMA,
        ],
    )
    out = gather(vt_lin, wtp, vid, wk, lc, wv, ww, misc)
    return out.reshape(B)


# speedup vs baseline: 12.5971x; 1.0710x over previous
import jax
import jax.numpy as jnp
from jax import lax
from jax.experimental import pallas as pl
from jax.experimental.pallas import tpu as pltpu
from jax.experimental.pallas import tpu_sc as plsc

N_VENDORS = 1000000
N_WEEKS = 520
NW_PAD = 528
EMB = 16
B = 16384
NWORK = 32
BPW = B // NWORK
NCHUNK = 4
CHUNK = BPW // NCHUNK

NGRP = 976
G_PER_W = 31
RAG_V0 = NGRP * 1024
RAG_W = N_VENDORS - RAG_V0
ROWS = 2 * (NGRP + 1)
KH_OFF = (NGRP + 1) * 8192


def _detile(vt_hbm, out_hbm, bufs, rag, rag2, sem_in, sem_out, sem_rag):
    nc = 2
    w = lax.axis_index("s") * nc + lax.axis_index("c")
    base = w * G_PER_W
    last = NGRP - 1

    def fire_in(i, p):
        g = jnp.minimum(base + i, last)
        for kh in range(2):
            for t in range(8):
                pltpu.async_copy(
                    vt_hbm.at[pl.ds(8 * kh, 8),
                              pl.ds(g * 1024 + t * 128, 128)],
                    bufs.at[p, kh, t], sem_in)

    def wait_in():
        for kh in range(2):
            pltpu.make_async_copy(
                out_hbm.at[0], bufs.at[0, kh], sem_in).wait()

    fire_in(0, 0)

    def body(i, carry):
        p = lax.rem(i, 2)
        g = jnp.minimum(base + i, last)
        wait_in()
        fire_in(i + 1, 1 - p)
        for kh in range(2):
            pltpu.async_copy(
                bufs.at[p, kh], out_hbm.at[kh * (NGRP + 1) + g], sem_out)
        for kh in range(2):
            pltpu.make_async_copy(
                out_hbm.at[0], bufs.at[0, kh], sem_out).wait()
        return carry

    lax.fori_loop(0, G_PER_W, body, 0)
    wait_in()

    @pl.when(w == 0)
    def _():
        cps = []
        for kh in range(2):
            for t in range(4):
                cps.append(pltpu.async_copy(
                    vt_hbm.at[pl.ds(8 * kh, 8),
                              pl.ds(RAG_V0 + t * 128, 128)],
                    rag.at[kh, t], sem_rag))
            cps.append(pltpu.async_copy(
                vt_hbm.at[pl.ds(8 * kh, 8), pl.ds(RAG_V0 + 512, 64)],
                rag2.at[kh], sem_rag))
        for cp in cps:
            cp.wait()
        for kh in range(2):
            for kl in range(8):
                for c in range(4):
                    rag[kh, 4, kl, pl.ds(c * EMB, EMB)] = (
                        rag2[kh, kl, pl.ds(c * EMB, EMB)])
        cps = []
        for kh in range(2):
            row = kh * (NGRP + 1) + NGRP
            cps.append(pltpu.async_copy(
                rag.at[kh], out_hbm.at[row], sem_rag))
        for cp in cps:
            cp.wait()


def _gather(vt_hbm, wtp_hbm, vid_hbm, wk_hbm, lc_hbm, wv_hbm, ww_hbm,
            misc_hbm, out_hbm, vid_v, idx_v, cols_v, wtab_v, wproj_v,
            wk_v, lc_v, wv_r, ww_r, misc_r, out_v, sem):
    nc = 2
    wid = lax.axis_index("s") * nc + lax.axis_index("c")

    pltpu.sync_copy(vid_hbm.at[pl.ds(wid * NCHUNK, NCHUNK)], vid_v)

    def expand(j, carry):
        for c in range(CHUNK // EMB):
            sl = pl.ds(c * EMB, EMB)
            v = vid_v[j, sl]
            vbase = (lax.shift_left(lax.shift_right_logical(v, 7), 10)
                     + lax.bitwise_and(v, 127))
            for k in range(EMB):
                off = (k // 8) * KH_OFF + (k % 8) * 128
                idx_v[k * NCHUNK + j, sl] = vbase + off
        return carry

    lax.fori_loop(0, NCHUNK, expand, 0)

    copies = []
    for k in range(EMB):
        for j in range(NCHUNK):
            copies.append(pltpu.async_copy(
                vt_hbm.at[idx_v.at[k * NCHUNK + j]], cols_v.at[k, j], sem))

    pltpu.sync_copy(wk_hbm.at[pl.ds(wid * NCHUNK, NCHUNK)], wk_v)
    pltpu.sync_copy(lc_hbm.at[pl.ds(wid * NCHUNK, NCHUNK)], lc_v)
    pltpu.sync_copy(wtp_hbm, wtab_v)
    pltpu.sync_copy(wv_hbm, wv_r)
    pltpu.sync_copy(ww_hbm, ww_r)
    pltpu.sync_copy(misc_hbm, misc_r)

    scale = misc_r[0] * misc_r[1]
    bias = misc_r[2]

    for c in range(NW_PAD // EMB):
        sl = pl.ds(c * EMB, EMB)
        acc = wtab_v[0, sl] * ww_r[0]
        for k in range(1, EMB):
            acc = acc + wtab_v[k, sl] * ww_r[k]
        wproj_v[c] = acc

    for cp in copies:
        cp.wait()

    def body(j, carry):
        for c in range(CHUNK // EMB):
            sl = pl.ds(c * EMB, EMB)
            wid_vec = wk_v[j, sl]
            acc0 = lc_v[j, sl] * scale + bias
            acc0 = acc0 + plsc.load_gather(
                wproj_v, [lax.shift_right_logical(wid_vec, 4),
                          lax.bitwise_and(wid_vec, 15)])
            acc1 = cols_v[0, j, sl] * wv_r[0]
            acc2 = cols_v[1, j, sl] * wv_r[1]
            acc3 = cols_v[2, j, sl] * wv_r[2]
            accs = [acc0, acc1, acc2, acc3]
            for k in range(3, EMB):
                a = accs[(k + 1) % 4]
                accs[(k + 1) % 4] = a + cols_v[k, j, sl] * wv_r[k]
            out_v[j, sl] = (accs[0] + accs[1]) + (accs[2] + accs[3])
        return carry

    lax.fori_loop(0, NCHUNK, body, 0)
    pltpu.sync_copy(out_v, out_hbm.at[pl.ds(wid * NCHUNK, NCHUNK)])


@jax.jit
def kernel(vendor_ids, week_ids, log_clicks, vendor_emb, week_emb, click_w,
           comb_W, comb_b):
    vt = vendor_emb.T
    wtp = jnp.pad(week_emb.T, ((0, 0), (0, NW_PAD - N_WEEKS)))
    vid = vendor_ids.reshape(NWORK * NCHUNK, CHUNK)
    wk = week_ids.reshape(NWORK * NCHUNK, CHUNK)
    lc = log_clicks.reshape(NWORK * NCHUNK, CHUNK)
    wv = jnp.broadcast_to(comb_W[0, 0:EMB, None], (EMB, EMB))
    ww = jnp.broadcast_to(comb_W[0, EMB:2 * EMB, None], (EMB, EMB))
    misc = jnp.broadcast_to(
        jnp.concatenate([comb_W[0, 2 * EMB:], click_w[0], comb_b])[:, None],
        (3, EMB))

    mesh = plsc.VectorSubcoreMesh(core_axis_name="c", subcore_axis_name="s")
    detile = pl.kernel(
        _detile, mesh=mesh,
        compiler_params=pltpu.CompilerParams(
            needs_layout_passes=False, use_tc_tiling_on_sc=True),
        out_type=jax.ShapeDtypeStruct((ROWS, 8, 8, 128), jnp.float32),
        scratch_types=[
            pltpu.VMEM((2, 2, 8, 8, 128), jnp.float32),
            pltpu.VMEM((2, 8, 8, 128), jnp.float32),
            pltpu.VMEM((2, 8, 64), jnp.float32),
            pltpu.SemaphoreType.DMA,
            pltpu.SemaphoreType.DMA,
            pltpu.SemaphoreType.DMA,
        ],
    )
    vt_lin = detile(vt).reshape(ROWS * 8192)

    mesh2 = plsc.VectorSubcoreMesh(core_axis_name="c", subcore_axis_name="s")
    gather = pl.kernel(
        _gather, mesh=mesh2,
        compiler_params=pltpu.CompilerParams(
            needs_layout_passes=False, use_tc_tiling_on_sc=False),
        out_type=jax.ShapeDtypeStruct((NWORK * NCHUNK, CHUNK), jnp.float32),
        scratch_types=[
            pltpu.VMEM((NCHUNK, CHUNK), jnp.int32),
            pltpu.VMEM((EMB * NCHUNK, CHUNK), jnp.int32),
            pltpu.VMEM((EMB, NCHUNK, CHUNK), jnp.float32),
            pltpu.VMEM((EMB, NW_PAD), jnp.float32),
            pltpu.VMEM((NW_PAD // EMB, EMB), jnp.float32),
            pltpu.VMEM((NCHUNK, CHUNK), jnp.int32),
            pltpu.VMEM((NCHUNK, CHUNK), jnp.float32),
            pltpu.VMEM((EMB, EMB), jnp.float32),
            pltpu.VMEM((EMB, EMB), jnp.float32),
            pltpu.VMEM((3, EMB), jnp.float32),
            pltpu.VMEM((NCHUNK, CHUNK), jnp.float32),
            pltpu.SemaphoreType.DMA,
        ],
    )
    out = gather(vt_lin, wtp, vid, wk, lc, wv, ww, misc)
    return out.reshape(B)
